# baseline (speedup 1.0000x reference)
import jax
import jax.numpy as jnp
from jax.experimental import pallas as pl
from jax.experimental.pallas import tpu as pltpu
from jax.experimental.pallas import tpu_sc as plsc


def kernel(patches, pos_table):
    B, N, D = patches.shape
    RB = 16
    W = 32
    HPW = N // (RB * W)

    mesh = plsc.VectorSubcoreMesh(core_axis_name="c", subcore_axis_name="s")

    @pl.kernel(out_type=jax.ShapeDtypeStruct((B, N, D), patches.dtype),
               mesh=mesh)
    def sc_kernel(p_hbm, t_hbm, o_hbm):
        def body(p_vmem, t_vmem, o_vmem):
            @pl.loop(0, RB)
            def _(r):
                @pl.loop(0, D, step=16)
                def _(i):
                    o_vmem.at[0, r, pl.ds(i, 16)][...] = (
                        p_vmem.at[0, r, pl.ds(i, 16)][...]
                        + t_vmem.at[r, pl.ds(i, 16)][...]
                    )

        pltpu.emit_pipeline(
            body,
            grid=(W, HPW, B),
            in_specs=[
                pl.BlockSpec((1, RB, D),
                             index_map=lambda c, h, b: (b, c * HPW + h, 0)),
                pl.BlockSpec((RB, D),
                             index_map=lambda c, h, b: (c * HPW + h, 0)),
            ],
            out_specs=[
                pl.BlockSpec((1, RB, D),
                             index_map=lambda c, h, b: (b, c * HPW + h, 0)),
            ],
            core_axis_name=("c", "s"),
            dimension_semantics=(pltpu.PARALLEL, pltpu.ARBITRARY,
                                 pltpu.ARBITRARY),
        )(p_hbm, t_hbm, o_hbm)

    return sc_kernel(patches, pos_table)


# speedup vs baseline: 4.4786x; 4.4786x over previous
import jax
import jax.numpy as jnp
from jax.experimental import pallas as pl


def _add_kernel(p_ref, t_ref, o_ref):
    o_ref[...] = p_ref[...] + t_ref[...]


def kernel(patches, pos_table):
    B, N, D = patches.shape
    BB, NB = 8, 512
    return pl.pallas_call(
        _add_kernel,
        grid=(B // BB, N // NB),
        in_specs=[
            pl.BlockSpec((BB, NB, D), lambda b, n: (b, n, 0)),
            pl.BlockSpec((NB, D), lambda b, n: (n, 0)),
        ],
        out_specs=pl.BlockSpec((BB, NB, D), lambda b, n: (b, n, 0)),
        out_shape=jax.ShapeDtypeStruct((B, N, D), patches.dtype),
    )(patches, pos_table)


# speedup vs baseline: 4.7786x; 1.0670x over previous
import jax
import jax.numpy as jnp
from jax.experimental import pallas as pl


def _add_kernel(p_ref, t_ref, o_ref):
    o_ref[...] = p_ref[...] + t_ref[...]


def kernel(patches, pos_table):
    B, N, D = patches.shape
    BB = 4
    return pl.pallas_call(
        _add_kernel,
        grid=(B // BB,),
        in_specs=[
            pl.BlockSpec((BB, N, D), lambda b: (b, 0, 0)),
            pl.BlockSpec((N, D), lambda b: (0, 0)),
        ],
        out_specs=pl.BlockSpec((BB, N, D), lambda b: (b, 0, 0)),
        out_shape=jax.ShapeDtypeStruct((B, N, D), patches.dtype),
    )(patches, pos_table)
